# Initial kernel scaffold; baseline (speedup 1.0000x reference)
import jax
import jax.numpy as jnp
from jax.experimental import pallas as pl


def kernel(bbox, x_tab, y_tab, h_tab, w_tab):
    raise NotImplementedError("write your pallas kernel here")


# speedup vs baseline: 2.7824x; 2.7824x over previous
import functools

import jax
import jax.numpy as jnp
from jax import lax
from jax.experimental import pallas as pl
from jax.experimental.pallas import tpu as pltpu
from jax.experimental.pallas import tpu_sc as plsc

_B, _S, _F, _D = 4, 8192, 6, 128
_NPOS = 1025
_N = _B * _S
_R = _N * _F
_L = 16
_NC, _NS = 2, 16
_NW = _NC * _NS
_PW = _R // _NW
_CH = 128
_NJ = _PW // _CH
_NV = _PW // _L


def _sc_body(bbox_hbm, tab_hbm, out_hbm, idx_v, rows_v, gsem):
    wid = lax.axis_index("s") * _NC + lax.axis_index("c")
    base = wid * _PW
    pltpu.sync_copy(bbox_hbm.at[pl.ds(base, _PW)], idx_v)

    def compute_idx(u, carry):
        p = u * _L + lax.iota(jnp.int32, _L)
        m = p % _F
        t = jnp.where(m < 4, m & 1, m - 2)
        idx_v[pl.ds(u * _L, _L)] = idx_v[pl.ds(u * _L, _L)] + t * _NPOS
        return carry

    lax.fori_loop(0, _NV, compute_idx, 0)

    def gather_one(j, carry):
        pltpu.async_copy(
            tab_hbm.at[idx_v.at[pl.ds(j * _CH, _CH)]], rows_v, gsem
        ).wait()
        pltpu.sync_copy(rows_v, out_hbm.at[pl.ds(base + j * _CH, _CH)])
        return carry

    lax.fori_loop(0, _NJ, gather_one, 0)


@jax.jit
def _sc_gather(bbox_flat, tab):
    mesh = plsc.VectorSubcoreMesh(core_axis_name="c", subcore_axis_name="s")
    return pl.kernel(
        _sc_body,
        mesh=mesh,
        out_type=jax.ShapeDtypeStruct((_R, _D), jnp.float32),
        scratch_types=[
            pltpu.VMEM((_PW,), jnp.int32),
            pltpu.VMEM((_CH, _D), jnp.float32),
            pltpu.SemaphoreType.DMA,
        ],
    )(bbox_flat, tab)


def kernel(bbox, x_tab, y_tab, h_tab, w_tab):
    bbox_flat = bbox.astype(jnp.int32).reshape(_R)
    tab = jnp.concatenate([x_tab, y_tab, h_tab, w_tab], axis=0)
    out = _sc_gather(bbox_flat, tab)
    return out.reshape(_B, _S, _F * _D)


# speedup vs baseline: 2.9546x; 1.0619x over previous
import functools

import jax
import jax.numpy as jnp
from jax import lax
from jax.experimental import pallas as pl
from jax.experimental.pallas import tpu as pltpu
from jax.experimental.pallas import tpu_sc as plsc

_B, _S, _F, _D = 4, 8192, 6, 128
_NPOS = 1025
_N = _B * _S
_R = _N * _F
_L = 16
_NC, _NS = 2, 16
_NW = _NC * _NS
_PW = _R // _NW
_CH = 128
_NJ = _PW // _CH
_NV = _PW // _L


def _sc_body(bbox_hbm, tab_hbm, out_hbm, idx_v, rows_v, gsem):
    wid = lax.axis_index("s") * _NC + lax.axis_index("c")
    base = wid * _PW
    pltpu.sync_copy(bbox_hbm.at[pl.ds(base, _PW)], idx_v)

    def compute_idx(u, carry):
        p = u * _L + lax.iota(jnp.int32, _L)
        m = p % _F
        t = jnp.where(m < 4, m & 1, m - 2)
        idx_v[pl.ds(u * _L, _L)] = idx_v[pl.ds(u * _L, _L)] + t * _NPOS
        return carry

    lax.fori_loop(0, _NV, compute_idx, 0)

    def start_gather(j, b):
        pltpu.make_async_copy(
            tab_hbm.at[idx_v.at[pl.ds(j * _CH, _CH)]], rows_v.at[b], gsem
        ).start()

    def wait_gather(b):
        pltpu.make_async_copy(
            tab_hbm.at[idx_v.at[pl.ds(0, _CH)]], rows_v.at[b], gsem
        ).wait()

    start_gather(0, 0)

    def chunk_pair(g, carry):
        for b in range(2):
            j = 2 * g + b
            wait_gather(b)

            @pl.when(j + 1 < _NJ)
            def _():
                start_gather(j + 1, 1 - b)

            pltpu.sync_copy(rows_v.at[b], out_hbm.at[pl.ds(base + j * _CH, _CH)])
        return carry

    lax.fori_loop(0, _NJ // 2, chunk_pair, 0)


@jax.jit
def _sc_gather(bbox_flat, tab):
    mesh = plsc.VectorSubcoreMesh(core_axis_name="c", subcore_axis_name="s")
    return pl.kernel(
        _sc_body,
        mesh=mesh,
        out_type=jax.ShapeDtypeStruct((_R, _D), jnp.float32),
        scratch_types=[
            pltpu.VMEM((_PW,), jnp.int32),
            pltpu.VMEM((2, _CH, _D), jnp.float32),
            pltpu.SemaphoreType.DMA,
        ],
    )(bbox_flat, tab)


def kernel(bbox, x_tab, y_tab, h_tab, w_tab):
    bbox_flat = bbox.astype(jnp.int32).reshape(_R)
    tab = jnp.concatenate([x_tab, y_tab, h_tab, w_tab], axis=0)
    out = _sc_gather(bbox_flat, tab)
    return out.reshape(_B, _S, _F * _D)


# speedup vs baseline: 3.6698x; 1.2421x over previous
import functools

import jax
import jax.numpy as jnp
from jax import lax
from jax.experimental import pallas as pl
from jax.experimental.pallas import tpu as pltpu
from jax.experimental.pallas import tpu_sc as plsc

_B, _S, _F, _D = 4, 8192, 6, 128
_NPOS = 1025
_N = _B * _S
_R = _N * _F
_L = 16
_NC, _NS = 2, 16
_NW = _NC * _NS
_PW = _R // _NW
_CH = 128
_NJ = _PW // _CH
_NV = _PW // _L


def _sc_body(bbox_hbm, tab_hbm, out_hbm, tab_sh, idx_v, rows_v, gsem):
    cid = lax.axis_index("c")
    sid = lax.axis_index("s")
    wid = sid * _NC + cid
    base = wid * _PW

    pltpu.sync_copy(tab_hbm.at[pl.ds(sid * 256, 256)], tab_sh.at[pl.ds(sid * 256, 256)])

    @pl.when(sid == 0)
    def _():
        pltpu.sync_copy(tab_hbm.at[pl.ds(4096, 4)], tab_sh.at[pl.ds(4096, 4)])

    pltpu.sync_copy(bbox_hbm.at[pl.ds(base, _PW)], idx_v)
    plsc.subcore_barrier()

    def compute_idx(u, carry):
        p = u * _L + lax.iota(jnp.int32, _L)
        m = p % _F
        t = jnp.where(m < 4, m & 1, m - 2)
        idx_v[pl.ds(u * _L, _L)] = idx_v[pl.ds(u * _L, _L)] + t * _NPOS
        return carry

    lax.fori_loop(0, _NV, compute_idx, 0)

    def start_gather(j, b):
        pltpu.make_async_copy(
            tab_sh.at[idx_v.at[pl.ds(j * _CH, _CH)]], rows_v.at[b], gsem
        ).start()

    def wait_gather(b):
        pltpu.make_async_copy(
            tab_sh.at[idx_v.at[pl.ds(0, _CH)]], rows_v.at[b], gsem
        ).wait()

    start_gather(0, 0)

    def chunk_pair(g, carry):
        for b in range(2):
            j = 2 * g + b
            wait_gather(b)

            @pl.when(j + 1 < _NJ)
            def _():
                start_gather(j + 1, 1 - b)

            pltpu.sync_copy(rows_v.at[b], out_hbm.at[pl.ds(base + j * _CH, _CH)])
        return carry

    lax.fori_loop(0, _NJ // 2, chunk_pair, 0)


@jax.jit
def _sc_gather(bbox_flat, tab):
    mesh = plsc.VectorSubcoreMesh(core_axis_name="c", subcore_axis_name="s")
    return pl.kernel(
        _sc_body,
        mesh=mesh,
        out_type=jax.ShapeDtypeStruct((_R, _D), jnp.float32),
        scratch_types=[
            pltpu.VMEM_SHARED((4 * _NPOS, _D), jnp.float32),
            pltpu.VMEM((_PW,), jnp.int32),
            pltpu.VMEM((2, _CH, _D), jnp.float32),
            pltpu.SemaphoreType.DMA,
        ],
    )(bbox_flat, tab)


def kernel(bbox, x_tab, y_tab, h_tab, w_tab):
    bbox_flat = bbox.astype(jnp.int32).reshape(_R)
    tab = jnp.concatenate([x_tab, y_tab, h_tab, w_tab], axis=0)
    out = _sc_gather(bbox_flat, tab)
    return out.reshape(_B, _S, _F * _D)


# speedup vs baseline: 8.0776x; 2.2011x over previous
import jax
import jax.numpy as jnp
from jax import lax
from jax.experimental import pallas as pl
from jax.experimental.pallas import tpu as pltpu
from jax.experimental.pallas import tpu_sc as plsc

_B, _S, _F, _D = 4, 8192, 6, 128
_NPOS = 1025
_L = 16
_NC, _NS = 2, 16
_NW = _NC * _NS
_TW = _B * _S // _NW
_CT = 32
_NK = _TW // _CT
_OFF = (0, _NPOS, 0, _NPOS, 2 * _NPOS, 3 * _NPOS)


def _sc_body(bbox_hbm, x_hbm, y_hbm, h_hbm, w_hbm, out_hbm,
             tab_sh, bb_v, idx_v, rows_v, gsem):
    cid = lax.axis_index("c")
    sid = lax.axis_index("s")
    wid = sid * _NC + cid
    bat = wid // (_S // _TW)
    s0 = (wid % (_S // _TW)) * _TW

    q = sid % 4
    tabs = (x_hbm, y_hbm, h_hbm, w_hbm)
    for f in range(4):
        @pl.when(sid // 4 == f)
        def _():
            pltpu.sync_copy(
                tabs[f].at[pl.ds(q * 256, 256)],
                tab_sh.at[pl.ds(f * _NPOS + q * 256, 256)],
            )

            @pl.when(q == 3)
            def _():
                pltpu.sync_copy(
                    tabs[f].at[pl.ds(1024, 1)],
                    tab_sh.at[pl.ds(f * _NPOS + 1024, 1)],
                )

    pltpu.sync_copy(bbox_hbm.at[pl.ds(wid * _TW * _F, _TW * _F)], bb_v)

    def build_idx(u, carry):
        t6 = (u * _L + lax.iota(jnp.int32, _L)) * _F
        for f in range(_F):
            v = plsc.load_gather(bb_v, [t6 + f])
            idx_v[f, pl.ds(u * _L, _L)] = v + _OFF[f]
        return carry

    lax.fori_loop(0, _TW // _L, build_idx, 0)
    plsc.subcore_barrier()

    def start_chunk(k, b2):
        tok0 = k * _CT
        for f in range(_F):
            pltpu.make_async_copy(
                tab_sh.at[idx_v.at[f, pl.ds(tok0, _CT)]],
                rows_v.at[b2, f],
                gsem,
            ).start()

    def wait_chunk(b2):
        for f in range(_F):
            pltpu.make_async_copy(
                tab_sh.at[idx_v.at[0, pl.ds(0, _CT)]],
                rows_v.at[b2, f],
                gsem,
            ).wait()

    start_chunk(0, 0)

    def chunk_pair(g, carry):
        for b2 in range(2):
            k = 2 * g + b2
            wait_chunk(b2)

            @pl.when(k + 1 < _NK)
            def _():
                start_chunk(k + 1, 1 - b2)

            for f in range(_F):
                pltpu.sync_copy(
                    rows_v.at[b2, f],
                    out_hbm.at[bat, pl.ds(s0 + k * _CT, _CT),
                               pl.ds(f * _D, _D)],
                )
        return carry

    lax.fori_loop(0, _NK // 2, chunk_pair, 0)


@jax.jit
def _sc_embed(bbox, x_tab, y_tab, h_tab, w_tab):
    mesh = plsc.VectorSubcoreMesh(core_axis_name="c", subcore_axis_name="s")
    return pl.kernel(
        _sc_body,
        mesh=mesh,
        compiler_params=pltpu.CompilerParams(needs_layout_passes=False),
        out_type=jax.ShapeDtypeStruct((_B, _S, _F * _D), jnp.float32),
        scratch_types=[
            pltpu.VMEM_SHARED((4 * _NPOS, _D), jnp.float32),
            pltpu.VMEM((_TW * _F,), jnp.int32),
            pltpu.VMEM((_F, _TW), jnp.int32),
            pltpu.VMEM((2, _F, _CT, _D), jnp.float32),
            pltpu.SemaphoreType.DMA,
        ],
    )(bbox, x_tab, y_tab, h_tab, w_tab)


def kernel(bbox, x_tab, y_tab, h_tab, w_tab):
    bbox_flat = bbox.astype(jnp.int32).reshape(_B * _S * _F)
    return _sc_embed(bbox_flat, x_tab, y_tab, h_tab, w_tab)


# speedup vs baseline: 10.4251x; 1.2906x over previous
import jax
import jax.numpy as jnp
from jax import lax
from jax.experimental import pallas as pl
from jax.experimental.pallas import tpu as pltpu
from jax.experimental.pallas import tpu_sc as plsc

_B, _S, _F, _D = 4, 8192, 6, 128
_NPOS = 1025
_L = 16
_NC, _NS = 2, 16
_NW = _NC * _NS
_TW = _B * _S // _NW
_CT = 32
_NK = _TW // _CT
_OFF = (0, _NPOS, 0, _NPOS, 2 * _NPOS, 3 * _NPOS)


def _sc_body(bbox_hbm, x_hbm, y_hbm, h_hbm, w_hbm, out_hbm,
             tab_sh, idx_v, rows_v, gsem):
    cid = lax.axis_index("c")
    sid = lax.axis_index("s")
    wid = sid * _NC + cid
    bat = wid // (_S // _TW)
    s0 = (wid % (_S // _TW)) * _TW

    q = sid % 4
    tabs = (x_hbm, y_hbm, h_hbm, w_hbm)
    for f in range(4):
        @pl.when(sid // 4 == f)
        def _():
            pltpu.sync_copy(
                tabs[f].at[pl.ds(q * 256, 256)],
                tab_sh.at[pl.ds(f * _NPOS + q * 256, 256)],
            )

            @pl.when(q == 3)
            def _():
                pltpu.sync_copy(
                    tabs[f].at[pl.ds(1024, 1)],
                    tab_sh.at[pl.ds(f * _NPOS + 1024, 1)],
                )

    for f in range(_F):
        pltpu.sync_copy(
            bbox_hbm.at[pl.ds(f * _B * _S + wid * _TW, _TW)],
            idx_v.at[pl.ds(f * _TW, _TW)],
        )

    def add_off(u, carry):
        for f in range(_F):
            if _OFF[f]:
                s = pl.ds(f * _TW + u * _L, _L)
                idx_v[s] = idx_v[s] + _OFF[f]
        return carry

    lax.fori_loop(0, _TW // _L, add_off, 0)
    plsc.subcore_barrier()

    def start_chunk(k, b2):
        tok0 = k * _CT
        for f in range(_F):
            pltpu.make_async_copy(
                tab_sh.at[idx_v.at[pl.ds(f * _TW + tok0, _CT)]],
                rows_v.at[b2, f],
                gsem,
            ).start()

    def wait_chunk(b2):
        for f in range(_F):
            pltpu.make_async_copy(
                tab_sh.at[idx_v.at[pl.ds(0, _CT)]],
                rows_v.at[b2, f],
                gsem,
            ).wait()

    start_chunk(0, 0)

    def chunk_pair(g, carry):
        for b2 in range(2):
            k = 2 * g + b2
            wait_chunk(b2)

            @pl.when(k + 1 < _NK)
            def _():
                start_chunk(k + 1, 1 - b2)

            for f in range(_F):
                pltpu.sync_copy(
                    rows_v.at[b2, f],
                    out_hbm.at[bat, pl.ds(s0 + k * _CT, _CT),
                               pl.ds(f * _D, _D)],
                )
        return carry

    lax.fori_loop(0, _NK // 2, chunk_pair, 0)


@jax.jit
def _sc_embed(bbox, x_tab, y_tab, h_tab, w_tab):
    mesh = plsc.VectorSubcoreMesh(core_axis_name="c", subcore_axis_name="s")
    return pl.kernel(
        _sc_body,
        mesh=mesh,
        compiler_params=pltpu.CompilerParams(needs_layout_passes=False),
        out_type=jax.ShapeDtypeStruct((_B, _S, _F * _D), jnp.float32),
        scratch_types=[
            pltpu.VMEM_SHARED((4 * _NPOS, _D), jnp.float32),
            pltpu.VMEM((_F * _TW,), jnp.int32),
            pltpu.VMEM((2, _F, _CT, _D), jnp.float32),
            pltpu.SemaphoreType.DMA,
        ],
    )(bbox, x_tab, y_tab, h_tab, w_tab)


def kernel(bbox, x_tab, y_tab, h_tab, w_tab):
    bbox_fm = bbox.astype(jnp.int32).transpose(2, 0, 1).reshape(_F * _B * _S)
    return _sc_embed(bbox_fm, x_tab, y_tab, h_tab, w_tab)


# speedup vs baseline: 10.8692x; 1.0426x over previous
import jax
import jax.numpy as jnp
from jax import lax
from jax.experimental import pallas as pl
from jax.experimental.pallas import tpu as pltpu
from jax.experimental.pallas import tpu_sc as plsc

_B, _S, _F, _D = 4, 8192, 6, 128
_NPOS = 1025
_L = 16
_NC, _NS = 2, 16
_NW = _NC * _NS
_TW = _B * _S // _NW
_CT = 128
_NK = _TW // _CT


def _sc_body(bbox_hbm, x_hbm, y_hbm, h_hbm, w_hbm, out_hbm,
             x_sh, y_sh, h_sh, w_sh, idx_v, rows_v, gsem):
    cid = lax.axis_index("c")
    sid = lax.axis_index("s")
    wid = sid * _NC + cid
    bat = wid // (_S // _TW)
    s0 = (wid % (_S // _TW)) * _TW

    q = sid % 4
    tabs_hbm = (x_hbm, y_hbm, h_hbm, w_hbm)
    tabs_sh = (x_sh, y_sh, h_sh, w_sh)
    for f in range(4):
        @pl.when(sid // 4 == f)
        def _():
            pltpu.sync_copy(
                tabs_hbm[f].at[pl.ds(q * 256, 256)],
                tabs_sh[f].at[pl.ds(q * 256, 256)],
            )

            @pl.when(q == 3)
            def _():
                pltpu.sync_copy(
                    tabs_hbm[f].at[pl.ds(1024, 1)],
                    tabs_sh[f].at[pl.ds(1024, 1)],
                )

    for f in range(_F):
        pltpu.sync_copy(
            bbox_hbm.at[pl.ds(f * _B * _S + wid * _TW, _TW)],
            idx_v.at[pl.ds(f * _TW, _TW)],
        )

    plsc.subcore_barrier()

    fsrc = (x_sh, y_sh, x_sh, y_sh, h_sh, w_sh)

    def start_unit(k, f, b2):
        pltpu.make_async_copy(
            fsrc[f].at[idx_v.at[pl.ds(f * _TW + k * _CT, _CT)]],
            rows_v.at[b2],
            gsem,
        ).start()

    def wait_unit(b2):
        pltpu.make_async_copy(
            fsrc[0].at[idx_v.at[pl.ds(0, _CT)]],
            rows_v.at[b2],
            gsem,
        ).wait()

    start_unit(0, 0, 0)

    def chunk(k, carry):
        for f in range(_F):
            b2 = f % 2
            wait_unit(b2)
            if f + 1 < _F:
                start_unit(k, f + 1, (f + 1) % 2)
            else:
                @pl.when(k + 1 < _NK)
                def _():
                    start_unit(k + 1, 0, 0)

            pltpu.sync_copy(
                rows_v.at[b2],
                out_hbm.at[bat, pl.ds(s0 + k * _CT, _CT), pl.ds(f * _D, _D)],
            )
        return carry

    lax.fori_loop(0, _NK, chunk, 0)


@jax.jit
def _sc_embed(bbox_fm, x_tab, y_tab, h_tab, w_tab):
    mesh = plsc.VectorSubcoreMesh(core_axis_name="c", subcore_axis_name="s")
    return pl.kernel(
        _sc_body,
        mesh=mesh,
        compiler_params=pltpu.CompilerParams(needs_layout_passes=False),
        out_type=jax.ShapeDtypeStruct((_B, _S, _F * _D), jnp.float32),
        scratch_types=[
            pltpu.VMEM_SHARED((_NPOS, _D), jnp.float32),
            pltpu.VMEM_SHARED((_NPOS, _D), jnp.float32),
            pltpu.VMEM_SHARED((_NPOS, _D), jnp.float32),
            pltpu.VMEM_SHARED((_NPOS, _D), jnp.float32),
            pltpu.VMEM((_F * _TW,), jnp.int32),
            pltpu.VMEM((2, _CT, _D), jnp.float32),
            pltpu.SemaphoreType.DMA,
        ],
    )(bbox_fm, x_tab, y_tab, h_tab, w_tab)


def kernel(bbox, x_tab, y_tab, h_tab, w_tab):
    bbox_fm = bbox.astype(jnp.int32).transpose(2, 0, 1).reshape(_F * _B * _S)
    return _sc_embed(bbox_fm, x_tab, y_tab, h_tab, w_tab)


# speedup vs baseline: 12.1710x; 1.1198x over previous
import jax
import jax.numpy as jnp
from jax import lax
from jax.experimental import pallas as pl
from jax.experimental.pallas import tpu as pltpu
from jax.experimental.pallas import tpu_sc as plsc

_B, _S, _F, _D = 4, 8192, 6, 128
_NPOS = 1025
_L = 16
_NC, _NS = 2, 16
_NW = _NC * _NS
_TW = _B * _S // _NW
_CT = 128
_NK = _TW // _CT


def _sc_body(bbox_hbm, x_hbm, y_hbm, h_hbm, w_hbm, out_hbm,
             x_sh, y_sh, h_sh, w_sh, idx_v, rows_v, gsem):
    cid = lax.axis_index("c")
    sid = lax.axis_index("s")
    wid = sid * _NC + cid
    bat = wid // (_S // _TW)
    s0 = (wid % (_S // _TW)) * _TW

    q = sid % 4
    tabs_hbm = (x_hbm, y_hbm, h_hbm, w_hbm)
    tabs_sh = (x_sh, y_sh, h_sh, w_sh)
    copies = []
    for f in range(4):
        @pl.when(sid // 4 == f)
        def _():
            pltpu.make_async_copy(
                tabs_hbm[f].at[pl.ds(q * 256, 256)],
                tabs_sh[f].at[pl.ds(q * 256, 256)],
                gsem,
            ).start()

            @pl.when(q == 3)
            def _():
                pltpu.make_async_copy(
                    tabs_hbm[f].at[pl.ds(1024, 1)],
                    tabs_sh[f].at[pl.ds(1024, 1)],
                    gsem,
                ).start()

    for f in range(_F):
        pltpu.make_async_copy(
            bbox_hbm.at[pl.ds(f * _B * _S + wid * _TW, _TW)],
            idx_v.at[pl.ds(f * _TW, _TW)],
            gsem,
        ).start()

    pltpu.make_async_copy(
        tabs_hbm[0].at[pl.ds(0, 256)], tabs_sh[0].at[pl.ds(0, 256)], gsem
    ).wait()

    @pl.when(q == 3)
    def _():
        pltpu.make_async_copy(
            tabs_hbm[0].at[pl.ds(1024, 1)], tabs_sh[0].at[pl.ds(1024, 1)], gsem
        ).wait()

    for f in range(_F):
        pltpu.make_async_copy(
            bbox_hbm.at[pl.ds(0, _TW)], idx_v.at[pl.ds(f * _TW, _TW)], gsem
        ).wait()

    plsc.subcore_barrier()

    fsrc = (x_sh, y_sh, x_sh, y_sh, h_sh, w_sh)

    def start_unit(k, f, b2):
        pltpu.make_async_copy(
            fsrc[f].at[idx_v.at[pl.ds(f * _TW + k * _CT, _CT)]],
            rows_v.at[b2],
            gsem,
        ).start()

    def wait_unit(b2):
        pltpu.make_async_copy(
            fsrc[0].at[idx_v.at[pl.ds(0, _CT)]],
            rows_v.at[b2],
            gsem,
        ).wait()

    for m in range(3):
        start_unit(0, m, m)

    def chunk_pair(g, carry):
        k0 = 2 * g
        for j in range(2):
            k = k0 + j
            for f in range(_F):
                b2 = (6 * j + f) % 4
                wait_unit(b2)
                fn = (f + 3) % 6
                kn = k + (1 if f + 3 >= 6 else 0)

                @pl.when(kn < _NK)
                def _():
                    start_unit(kn, fn, (6 * j + f + 3) % 4)

                pltpu.sync_copy(
                    rows_v.at[b2],
                    out_hbm.at[bat, pl.ds(s0 + k * _CT, _CT),
                               pl.ds(f * _D, _D)],
                )
        return carry

    lax.fori_loop(0, _NK // 2, chunk_pair, 0)


@jax.jit
def _sc_embed(bbox_fm, x_tab, y_tab, h_tab, w_tab):
    mesh = plsc.VectorSubcoreMesh(core_axis_name="c", subcore_axis_name="s")
    return pl.kernel(
        _sc_body,
        mesh=mesh,
        compiler_params=pltpu.CompilerParams(needs_layout_passes=False),
        out_type=jax.ShapeDtypeStruct((_B, _S, _F * _D), jnp.float32),
        scratch_types=[
            pltpu.VMEM_SHARED((_NPOS, _D), jnp.float32),
            pltpu.VMEM_SHARED((_NPOS, _D), jnp.float32),
            pltpu.VMEM_SHARED((_NPOS, _D), jnp.float32),
            pltpu.VMEM_SHARED((_NPOS, _D), jnp.float32),
            pltpu.VMEM((_F * _TW,), jnp.int32),
            pltpu.VMEM((4, _CT, _D), jnp.float32),
            pltpu.SemaphoreType.DMA,
        ],
    )(bbox_fm, x_tab, y_tab, h_tab, w_tab)


def kernel(bbox, x_tab, y_tab, h_tab, w_tab):
    bbox_fm = bbox.astype(jnp.int32).transpose(2, 0, 1).reshape(_F * _B * _S)
    return _sc_embed(bbox_fm, x_tab, y_tab, h_tab, w_tab)
